# baseline (speedup 1.0000x reference)
import jax
import jax.numpy as jnp
from jax.experimental import pallas as pl
from jax.experimental.pallas import tpu as pltpu

D_MODEL = 1024
P = 128
D_INNER = 2048
B = 2
L = 4096
BL = B * L

BF = jnp.bfloat16
F32 = jnp.float32


def _rmsnorm(x, w):
    var = jnp.mean(x * x, axis=-1, keepdims=True)
    return x * jax.lax.rsqrt(var + 1e-5) * w


def _proj_in_compute(x, nw, win, bmt, u_ref, z_ref, bu_ref, slot):
    h = _rmsnorm(x, nw)
    xz = jnp.dot(h, win, preferred_element_type=F32)
    xs = xz[:, :D_INNER]
    z = xz[:, D_INNER:]
    u = xs * jax.nn.sigmoid(xs)
    u_ref[...] = u.astype(BF)
    z_ref[...] = z.astype(BF)
    bu_ref[slot] = jnp.dot(u, bmt, preferred_element_type=F32)


def _proj_out_compute(x, u, z, xs, cmt, dv, wout):
    y = jnp.dot(xs, cmt, preferred_element_type=F32)
    y = y + dv * u.astype(F32)
    zf = z.astype(F32)
    g = y * (zf * jax.nn.sigmoid(zf))
    return x + jnp.dot(g, wout, preferred_element_type=F32)


def _mat2_mul(a, b):
    a11, a12, a21, a22 = a
    b11, b12, b21, b22 = b
    return (a11 * b11 + a12 * b21, a11 * b12 + a12 * b22,
            a21 * b11 + a22 * b21, a21 * b12 + a22 * b22)


def _scan_chunk(bu, a_vec, st_vec, chunk, nc, c, c1_ref, c2_ref, xs_ref):
    A = jnp.maximum(a_vec, 0.0)
    dt = jax.nn.sigmoid(st_vec)
    S = 1.0 / (1.0 + dt * dt * A)
    m = (S, -dt * A * S, dt * S, 1.0 - dt * dt * A * S)
    alpha = dt * S
    beta = dt * dt * S

    @pl.when(c % nc == 0)
    def _():
        c1_ref[...] = jnp.zeros((1, P), F32)
        c2_ref[...] = jnp.zeros((1, P), F32)

    b1 = alpha * bu
    b2 = beta * bu

    p = m
    s = 1
    while s < chunk:
        zpad = jnp.zeros((s, P), F32)
        sh1 = jnp.concatenate([zpad, b1[: chunk - s]], axis=0)
        sh2 = jnp.concatenate([zpad, b2[: chunk - s]], axis=0)
        p11, p12, p21, p22 = p
        b1 = b1 + p11 * sh1 + p12 * sh2
        b2 = b2 + p21 * sh1 + p22 * sh2
        s *= 2
        if s < chunk:
            p = _mat2_mul(p, p)

    t11, t12, t21, t22 = m
    pw = m
    n = 1
    while n < chunk:
        pw11, pw12, pw21, pw22 = pw
        n11 = t11 * pw11 + t12 * pw21
        n12 = t11 * pw12 + t12 * pw22
        n21 = t21 * pw11 + t22 * pw21
        n22 = t21 * pw12 + t22 * pw22
        t11 = jnp.concatenate([t11, n11], axis=0)
        t12 = jnp.concatenate([t12, n12], axis=0)
        t21 = jnp.concatenate([t21, n21], axis=0)
        t22 = jnp.concatenate([t22, n22], axis=0)
        n *= 2
        if n < chunk:
            pw = _mat2_mul(pw, pw)

    c1 = c1_ref[...]
    c2 = c2_ref[...]
    b1 = b1 + t11 * c1 + t12 * c2
    b2 = b2 + t21 * c1 + t22 * c2

    c1_ref[...] = b1[chunk - 1 : chunk]
    c2_ref[...] = b2[chunk - 1 : chunk]
    xs_ref[...] = b2


ROWS1 = 512
NT1 = BL // ROWS1
NC1 = L // ROWS1


def _c1_body(x_ref, nw_ref, win_ref, bmt_ref, a_ref, st_ref,
             u_ref, z_ref, xs_ref, bu_s, c1_s, c2_s):
    i = pl.program_id(0)

    @pl.when(i < NT1)
    def _():
        _proj_in_compute(x_ref[...], nw_ref[...], win_ref[...], bmt_ref[...],
                         u_ref, z_ref, bu_s, i % 2)

    @pl.when(i > 0)
    def _():
        c = i - 1
        _scan_chunk(bu_s[(i - 1) % 2], a_ref[...], st_ref[...],
                    ROWS1, NC1, c, c1_s, c2_s, xs_ref)


def _call1(x2d, norm_w, Win, BmatT, A_p, steps):
    return pl.pallas_call(
        _c1_body,
        grid=(NT1 + 1,),
        in_specs=[
            pl.BlockSpec((ROWS1, D_MODEL),
                         lambda i: (jnp.minimum(i, NT1 - 1), 0)),
            pl.BlockSpec((1, D_MODEL), lambda i: (0, 0)),
            pl.BlockSpec((D_MODEL, 2 * D_INNER), lambda i: (0, 0)),
            pl.BlockSpec((D_INNER, P), lambda i: (0, 0)),
            pl.BlockSpec((1, P), lambda i: (0, 0)),
            pl.BlockSpec((1, P), lambda i: (0, 0)),
        ],
        out_specs=[
            pl.BlockSpec((ROWS1, D_INNER),
                         lambda i: (jnp.minimum(i, NT1 - 1), 0)),
            pl.BlockSpec((ROWS1, D_INNER),
                         lambda i: (jnp.minimum(i, NT1 - 1), 0)),
            pl.BlockSpec((ROWS1, P),
                         lambda i: (jnp.maximum(i - 1, 0), 0)),
        ],
        out_shape=[
            jax.ShapeDtypeStruct((BL, D_INNER), BF),
            jax.ShapeDtypeStruct((BL, D_INNER), BF),
            jax.ShapeDtypeStruct((BL, P), F32),
        ],
        scratch_shapes=[
            pltpu.VMEM((2, ROWS1, P), F32),
            pltpu.VMEM((1, P), F32),
            pltpu.VMEM((1, P), F32),
        ],
    )(x2d, norm_w.reshape(1, D_MODEL), Win, BmatT,
      A_p.reshape(1, P), steps.reshape(1, P))


ROWS2 = 256
NT2 = BL // ROWS2
NC2 = L // ROWS2


def _c2_body(x_ref, u_ref, z_ref, xs_ref, cmt_ref, dv_ref, wout_ref,
             nw_ref, win_ref, bmt_ref, a_ref, st_ref,
             x1_ref, u1_ref, z1_ref, xs1_ref, x1_s, bu_s, c1_s, c2_s):
    i = pl.program_id(0)

    @pl.when(i < NT2)
    def _():
        o = _proj_out_compute(x_ref[...], u_ref[...], z_ref[...], xs_ref[...],
                              cmt_ref[...], dv_ref[...], wout_ref[...])
        x1_ref[...] = o
        x1_s[i % 2] = o

    @pl.when((i > 0) & (i <= NT2))
    def _():
        _proj_in_compute(x1_s[(i - 1) % 2], nw_ref[...], win_ref[...],
                         bmt_ref[...], u1_ref, z1_ref, bu_s, (i - 1) % 2)

    @pl.when(i > 1)
    def _():
        c = i - 2
        _scan_chunk(bu_s[(i - 2) % 2], a_ref[...], st_ref[...],
                    ROWS2, NC2, c, c1_s, c2_s, xs1_ref)


def _call2(x2d, u, z, xs2d, CmatT, Dvec, Wout, norm_w1, Win1, BmatT1,
           A_p, steps):
    lo = lambda i: (jnp.minimum(i, NT2 - 1), 0)
    m1 = lambda i: (jnp.clip(i - 1, 0, NT2 - 1), 0)
    hi = lambda i: (jnp.clip(i - 2, 0, NT2 - 1), 0)
    full = lambda i: (0, 0)
    return pl.pallas_call(
        _c2_body,
        grid=(NT2 + 2,),
        in_specs=[
            pl.BlockSpec((ROWS2, D_MODEL), lo),
            pl.BlockSpec((ROWS2, D_INNER), lo),
            pl.BlockSpec((ROWS2, D_INNER), lo),
            pl.BlockSpec((ROWS2, P), lo),
            pl.BlockSpec((P, D_INNER), full),
            pl.BlockSpec((1, D_INNER), full),
            pl.BlockSpec((D_INNER, D_MODEL), full),
            pl.BlockSpec((1, D_MODEL), full),
            pl.BlockSpec((D_MODEL, 2 * D_INNER), full),
            pl.BlockSpec((D_INNER, P), full),
            pl.BlockSpec((1, P), full),
            pl.BlockSpec((1, P), full),
        ],
        out_specs=[
            pl.BlockSpec((ROWS2, D_MODEL), lo),
            pl.BlockSpec((ROWS2, D_INNER), m1),
            pl.BlockSpec((ROWS2, D_INNER), m1),
            pl.BlockSpec((ROWS2, P), hi),
        ],
        out_shape=[
            jax.ShapeDtypeStruct((BL, D_MODEL), F32),
            jax.ShapeDtypeStruct((BL, D_INNER), BF),
            jax.ShapeDtypeStruct((BL, D_INNER), BF),
            jax.ShapeDtypeStruct((BL, P), F32),
        ],
        scratch_shapes=[
            pltpu.VMEM((2, ROWS2, D_MODEL), F32),
            pltpu.VMEM((2, ROWS2, P), F32),
            pltpu.VMEM((1, P), F32),
            pltpu.VMEM((1, P), F32),
        ],
    )(x2d, u, z, xs2d, CmatT, Dvec.reshape(1, D_INNER), Wout,
      norm_w1.reshape(1, D_MODEL), Win1, BmatT1,
      A_p.reshape(1, P), steps.reshape(1, P))


ROWS3 = 512
NT3 = BL // ROWS3


def _c3_body(x_ref, u_ref, z_ref, xs_ref, cmt_ref, dv_ref, wout_ref,
             fw_ref, o_ref, g_s):
    i = pl.program_id(0)

    @pl.when(i < NT3)
    def _():
        y = jnp.dot(xs_ref[...], cmt_ref[...], preferred_element_type=F32)
        y = y + dv_ref[...] * u_ref[...].astype(F32)
        zf = z_ref[...].astype(F32)
        g_s[i % 2] = y * (zf * jax.nn.sigmoid(zf))

    @pl.when(i > 0)
    def _():
        o = x_ref[...] + jnp.dot(g_s[(i - 1) % 2], wout_ref[...],
                                 preferred_element_type=F32)
        o_ref[...] = _rmsnorm(o, fw_ref[...])


def _call3(x2d, u, z, xs2d, CmatT, Dvec, Wout, final_w):
    lo3 = lambda i: (jnp.minimum(i, NT3 - 1), 0)
    m13 = lambda i: (jnp.maximum(i - 1, 0), 0)
    return pl.pallas_call(
        _c3_body,
        grid=(NT3 + 1,),
        in_specs=[
            pl.BlockSpec((ROWS3, D_MODEL), m13),
            pl.BlockSpec((ROWS3, D_INNER), lo3),
            pl.BlockSpec((ROWS3, D_INNER), lo3),
            pl.BlockSpec((ROWS3, P), lo3),
            pl.BlockSpec((P, D_INNER), lambda i: (0, 0)),
            pl.BlockSpec((1, D_INNER), lambda i: (0, 0)),
            pl.BlockSpec((D_INNER, D_MODEL), lambda i: (0, 0)),
            pl.BlockSpec((1, D_MODEL), lambda i: (0, 0)),
        ],
        out_specs=pl.BlockSpec((ROWS3, D_MODEL), m13),
        out_shape=jax.ShapeDtypeStruct((BL, D_MODEL), F32),
        scratch_shapes=[pltpu.VMEM((2, ROWS3, D_INNER), F32)],
    )(x2d, u, z, xs2d, CmatT, Dvec.reshape(1, D_INNER), Wout,
      final_w.reshape(1, D_MODEL))


def kernel(x, norm_w_0, Win_0, A_0, steps_0, Bmat_0, Cmat_0, Dvec_0, Wout_0,
           norm_w_1, Win_1, A_1, steps_1, Bmat_1, Cmat_1, Dvec_1, Wout_1,
           final_norm_w):
    x2 = x.reshape(BL, D_MODEL)
    win0 = Win_0
    win1 = Win_1
    bmt0 = Bmat_0.T
    bmt1 = Bmat_1.T
    cmt0 = Cmat_0.T
    cmt1 = Cmat_1.T
    wout0 = Wout_0
    wout1 = Wout_1

    u0, z0, xs0 = _call1(x2, norm_w_0, win0, bmt0, A_0, steps_0)
    x1, u1, z1, xs1 = _call2(x2, u0, z0, xs0, cmt0, Dvec_0, wout0,
                             norm_w_1, win1, bmt1, A_1, steps_1)
    out = _call3(x1, u1, z1, xs1, cmt1, Dvec_1, wout1, final_norm_w)
    return out.reshape(B, L, D_MODEL)


# speedup vs baseline: 1.0217x; 1.0217x over previous
import jax
import jax.numpy as jnp
from jax.experimental import pallas as pl
from jax.experimental.pallas import tpu as pltpu

D_MODEL = 1024
P = 128
D_INNER = 2048
B = 2
L = 4096
BL = B * L

BF = jnp.bfloat16
F32 = jnp.float32


def _rmsnorm(x, w):
    var = jnp.mean(x * x, axis=-1, keepdims=True)
    return x * jax.lax.rsqrt(var + 1e-5) * w


def _proj_in_compute(x, nw, win, bmt, u_ref, z_ref, bu_ref, slot):
    h = _rmsnorm(x, nw)
    xz = jnp.dot(h, win, preferred_element_type=F32)
    xs = xz[:, :D_INNER]
    z = xz[:, D_INNER:]
    u = xs * jax.nn.sigmoid(xs)
    u_ref[...] = u.astype(BF)
    z_ref[...] = z.astype(BF)
    bu_ref[slot] = jnp.dot(u, bmt, preferred_element_type=F32)


def _proj_out_compute(x, u, z, xs, cmt, dv, wout):
    y = jnp.dot(xs, cmt, preferred_element_type=F32)
    y = y + dv * u.astype(F32)
    zf = z.astype(F32)
    g = y * (zf * jax.nn.sigmoid(zf))
    return x + jnp.dot(g, wout, preferred_element_type=F32)


def _mat2_mul(a, b):
    a11, a12, a21, a22 = a
    b11, b12, b21, b22 = b
    return (a11 * b11 + a12 * b21, a11 * b12 + a12 * b22,
            a21 * b11 + a22 * b21, a21 * b12 + a22 * b22)


def _scan_chunk(bu, a_vec, st_vec, chunk, nc, c, c1_ref, c2_ref, xs_ref):
    A = jnp.maximum(a_vec, 0.0)
    dt = jax.nn.sigmoid(st_vec)
    S = 1.0 / (1.0 + dt * dt * A)
    m = (S, -dt * A * S, dt * S, 1.0 - dt * dt * A * S)
    alpha = dt * S
    beta = dt * dt * S

    @pl.when(c % nc == 0)
    def _():
        c1_ref[...] = jnp.zeros((1, P), F32)
        c2_ref[...] = jnp.zeros((1, P), F32)

    b1 = alpha * bu
    b2 = beta * bu

    p = m
    s = 1
    while s < chunk:
        zpad = jnp.zeros((s, P), F32)
        sh1 = jnp.concatenate([zpad, b1[: chunk - s]], axis=0)
        sh2 = jnp.concatenate([zpad, b2[: chunk - s]], axis=0)
        p11, p12, p21, p22 = p
        b1 = b1 + p11 * sh1 + p12 * sh2
        b2 = b2 + p21 * sh1 + p22 * sh2
        s *= 2
        if s < chunk:
            p = _mat2_mul(p, p)

    t11, t12, t21, t22 = m
    pw = m
    n = 1
    while n < chunk:
        pw11, pw12, pw21, pw22 = pw
        n11 = t11 * pw11 + t12 * pw21
        n12 = t11 * pw12 + t12 * pw22
        n21 = t21 * pw11 + t22 * pw21
        n22 = t21 * pw12 + t22 * pw22
        t11 = jnp.concatenate([t11, n11], axis=0)
        t12 = jnp.concatenate([t12, n12], axis=0)
        t21 = jnp.concatenate([t21, n21], axis=0)
        t22 = jnp.concatenate([t22, n22], axis=0)
        n *= 2
        if n < chunk:
            pw = _mat2_mul(pw, pw)

    c1 = c1_ref[...]
    c2 = c2_ref[...]
    b1 = b1 + t11 * c1 + t12 * c2
    b2 = b2 + t21 * c1 + t22 * c2

    c1_ref[...] = b1[chunk - 1 : chunk]
    c2_ref[...] = b2[chunk - 1 : chunk]
    xs_ref[...] = b2


ROWS1 = 512
NT1 = BL // ROWS1
NC1 = L // ROWS1


def _c1_body(x_ref, nw_ref, win_ref, bmt_ref, a_ref, st_ref,
             u_ref, z_ref, xs_ref, bu_s, c1_s, c2_s):
    i = pl.program_id(0)

    @pl.when(i < NT1)
    def _():
        _proj_in_compute(x_ref[...], nw_ref[...], win_ref[...], bmt_ref[...],
                         u_ref, z_ref, bu_s, i % 2)

    @pl.when(i > 0)
    def _():
        c = i - 1
        _scan_chunk(bu_s[(i - 1) % 2], a_ref[...], st_ref[...],
                    ROWS1, NC1, c, c1_s, c2_s, xs_ref)


def _call1(x2d, norm_w, Win, BmatT, A_p, steps):
    return pl.pallas_call(
        _c1_body,
        grid=(NT1 + 1,),
        in_specs=[
            pl.BlockSpec((ROWS1, D_MODEL),
                         lambda i: (jnp.minimum(i, NT1 - 1), 0)),
            pl.BlockSpec((1, D_MODEL), lambda i: (0, 0)),
            pl.BlockSpec((D_MODEL, 2 * D_INNER), lambda i: (0, 0)),
            pl.BlockSpec((D_INNER, P), lambda i: (0, 0)),
            pl.BlockSpec((1, P), lambda i: (0, 0)),
            pl.BlockSpec((1, P), lambda i: (0, 0)),
        ],
        out_specs=[
            pl.BlockSpec((ROWS1, D_INNER),
                         lambda i: (jnp.minimum(i, NT1 - 1), 0)),
            pl.BlockSpec((ROWS1, D_INNER),
                         lambda i: (jnp.minimum(i, NT1 - 1), 0)),
            pl.BlockSpec((ROWS1, P),
                         lambda i: (jnp.maximum(i - 1, 0), 0)),
        ],
        out_shape=[
            jax.ShapeDtypeStruct((BL, D_INNER), BF),
            jax.ShapeDtypeStruct((BL, D_INNER), BF),
            jax.ShapeDtypeStruct((BL, P), F32),
        ],
        scratch_shapes=[
            pltpu.VMEM((2, ROWS1, P), F32),
            pltpu.VMEM((1, P), F32),
            pltpu.VMEM((1, P), F32),
        ],
    )(x2d, norm_w.reshape(1, D_MODEL), Win, BmatT,
      A_p.reshape(1, P), steps.reshape(1, P))


ROWS2 = 512
NT2 = BL // ROWS2
NC2 = L // ROWS2


def _c2_body(x_ref, u_ref, z_ref, xs_ref, cmt_ref, dv_ref, wout_ref,
             nw_ref, win_ref, bmt_ref, a_ref, st_ref,
             x1_ref, u1_ref, z1_ref, xs1_ref, bu_s, c1_s, c2_s):
    i = pl.program_id(0)

    @pl.when(i < NT2)
    def _():
        o = _proj_out_compute(x_ref[...], u_ref[...], z_ref[...], xs_ref[...],
                              cmt_ref[...], dv_ref[...], wout_ref[...])
        x1_ref[...] = o
        _proj_in_compute(o, nw_ref[...], win_ref[...], bmt_ref[...],
                         u1_ref, z1_ref, bu_s, i % 2)

    @pl.when(i > 0)
    def _():
        c = i - 1
        _scan_chunk(bu_s[(i - 1) % 2], a_ref[...], st_ref[...],
                    ROWS2, NC2, c, c1_s, c2_s, xs1_ref)


def _call2(x2d, u, z, xs2d, CmatT, Dvec, Wout, norm_w1, Win1, BmatT1,
           A_p, steps):
    lo = lambda i: (jnp.minimum(i, NT2 - 1), 0)
    hi = lambda i: (jnp.maximum(i - 1, 0), 0)
    full = lambda i: (0, 0)
    return pl.pallas_call(
        _c2_body,
        grid=(NT2 + 1,),
        in_specs=[
            pl.BlockSpec((ROWS2, D_MODEL), lo),
            pl.BlockSpec((ROWS2, D_INNER), lo),
            pl.BlockSpec((ROWS2, D_INNER), lo),
            pl.BlockSpec((ROWS2, P), lo),
            pl.BlockSpec((P, D_INNER), full),
            pl.BlockSpec((1, D_INNER), full),
            pl.BlockSpec((D_INNER, D_MODEL), full),
            pl.BlockSpec((1, D_MODEL), full),
            pl.BlockSpec((D_MODEL, 2 * D_INNER), full),
            pl.BlockSpec((D_INNER, P), full),
            pl.BlockSpec((1, P), full),
            pl.BlockSpec((1, P), full),
        ],
        out_specs=[
            pl.BlockSpec((ROWS2, D_MODEL), lo),
            pl.BlockSpec((ROWS2, D_INNER), lo),
            pl.BlockSpec((ROWS2, D_INNER), lo),
            pl.BlockSpec((ROWS2, P), hi),
        ],
        out_shape=[
            jax.ShapeDtypeStruct((BL, D_MODEL), F32),
            jax.ShapeDtypeStruct((BL, D_INNER), BF),
            jax.ShapeDtypeStruct((BL, D_INNER), BF),
            jax.ShapeDtypeStruct((BL, P), F32),
        ],
        scratch_shapes=[
            pltpu.VMEM((2, ROWS2, P), F32),
            pltpu.VMEM((1, P), F32),
            pltpu.VMEM((1, P), F32),
        ],
    )(x2d, u, z, xs2d, CmatT, Dvec.reshape(1, D_INNER), Wout,
      norm_w1.reshape(1, D_MODEL), Win1, BmatT1,
      A_p.reshape(1, P), steps.reshape(1, P))


ROWS3 = 512
NT3 = BL // ROWS3


def _c3_body(x_ref, u_ref, z_ref, xs_ref, cmt_ref, dv_ref, wout_ref,
             fw_ref, o_ref):
    o = _proj_out_compute(x_ref[...], u_ref[...], z_ref[...], xs_ref[...],
                          cmt_ref[...], dv_ref[...], wout_ref[...])
    o_ref[...] = _rmsnorm(o, fw_ref[...])


def _call3(x2d, u, z, xs2d, CmatT, Dvec, Wout, final_w):
    return pl.pallas_call(
        _c3_body,
        grid=(NT3,),
        in_specs=[
            pl.BlockSpec((ROWS3, D_MODEL), lambda i: (i, 0)),
            pl.BlockSpec((ROWS3, D_INNER), lambda i: (i, 0)),
            pl.BlockSpec((ROWS3, D_INNER), lambda i: (i, 0)),
            pl.BlockSpec((ROWS3, P), lambda i: (i, 0)),
            pl.BlockSpec((P, D_INNER), lambda i: (0, 0)),
            pl.BlockSpec((1, D_INNER), lambda i: (0, 0)),
            pl.BlockSpec((D_INNER, D_MODEL), lambda i: (0, 0)),
            pl.BlockSpec((1, D_MODEL), lambda i: (0, 0)),
        ],
        out_specs=pl.BlockSpec((ROWS3, D_MODEL), lambda i: (i, 0)),
        out_shape=jax.ShapeDtypeStruct((BL, D_MODEL), F32),
    )(x2d, u, z, xs2d, CmatT, Dvec.reshape(1, D_INNER), Wout,
      final_w.reshape(1, D_MODEL))


def kernel(x, norm_w_0, Win_0, A_0, steps_0, Bmat_0, Cmat_0, Dvec_0, Wout_0,
           norm_w_1, Win_1, A_1, steps_1, Bmat_1, Cmat_1, Dvec_1, Wout_1,
           final_norm_w):
    x2 = x.reshape(BL, D_MODEL)
    win0 = Win_0
    win1 = Win_1
    bmt0 = Bmat_0.T
    bmt1 = Bmat_1.T
    cmt0 = Cmat_0.T
    cmt1 = Cmat_1.T
    wout0 = Wout_0
    wout1 = Wout_1

    u0, z0, xs0 = _call1(x2, norm_w_0, win0, bmt0, A_0, steps_0)
    x1, u1, z1, xs1 = _call2(x2, u0, z0, xs0, cmt0, Dvec_0, wout0,
                             norm_w_1, win1, bmt1, A_1, steps_1)
    out = _call3(x1, u1, z1, xs1, cmt1, Dvec_1, wout1, final_norm_w)
    return out.reshape(B, L, D_MODEL)
